# baseline (speedup 1.0000x reference)
import jax
import jax.numpy as jnp
from jax import lax
from jax.experimental import pallas as pl
from jax.experimental.pallas import tpu as pltpu

RADIUS, DECAY_RADIUS = 0.15, 0.3
EPS_BN = 1e-5
B, N = 4, 1024
C_IN, C_OUT, C_BYP = 64, 64, 32
C6 = 6 * C_BYP
TI = 256
BN_CNT = B * N
_R2 = RADIUS * RADIUS
_DR2 = DECAY_RADIUS * DECAY_RADIUS
_C1 = 1.0 / (_DR2 - _R2)
_C0 = _DR2 * _C1
_SIGN = -2147483648


def _bn_fold(x, gamma, beta):
    s1 = jnp.sum(x, axis=0, keepdims=True)
    s2 = jnp.sum(x * x, axis=0, keepdims=True)
    mean = s1 * (1.0 / BN_CNT)
    var = s2 * (1.0 / BN_CNT) - mean * mean
    a = gamma * lax.rsqrt(var + EPS_BN)
    return a, beta - mean * a


def _copysign_pos(u, d):
    ui = lax.bitcast_convert_type(u, jnp.int32)
    di = lax.bitcast_convert_type(d, jnp.int32)
    return lax.bitcast_convert_type(ui | (di & _SIGN), jnp.float32)


def _k_all(feat_ref, xyz_ref, xyzt_ref, wf_ref, bf_ref, wb_ref, gb_ref, beb_ref,
           wag_ref, bag_ref, g1_ref, b1_ref, g2_ref, b2_ref, out_ref, ag_scr):
    feat = feat_ref[...]
    self_feat = jnp.dot(feat, wf_ref[...], preferred_element_type=jnp.float32) + bf_ref[...]
    mut = jnp.dot(feat, wb_ref[...], preferred_element_type=jnp.float32)

    a_b, sh_b = _bn_fold(mut, gb_ref[...], beb_ref[...])

    ones_n = jnp.ones((N, C_BYP), dtype=jnp.float32)
    for b in range(B):
        g = jnp.maximum(mut[b * N:(b + 1) * N, :] * a_b + sh_b, 0.0)
        gs = [g[:, (2 * ax) * C_BYP:(2 * ax + 1) * C_BYP]
              + g[:, (2 * ax + 1) * C_BYP:(2 * ax + 2) * C_BYP] for ax in range(3)]
        gd = [g[:, (2 * ax) * C_BYP:(2 * ax + 1) * C_BYP]
              - g[:, (2 * ax + 1) * C_BYP:(2 * ax + 2) * C_BYP] for ax in range(3)]
        cj = [xyz_ref[b, :, ax:ax + 1] for ax in range(3)]
        sj = [c * c for c in cj]
        f2 = sj[0] * gs[0] + sj[1] * gs[1] + sj[2] * gs[2]
        bu = jnp.concatenate(
            [f2, cj[0] * gs[0], cj[1] * gs[1], cj[2] * gs[2],
             gs[0], gs[1], gs[2]], axis=1)
        bm = [jnp.concatenate([sj[ax] * gd[ax], cj[ax] * gd[ax], gd[ax]], axis=1)
              for ax in range(3)]
        xj = xyzt_ref[b]

        def body(i, _):
            xi = xyz_ref[b, pl.ds(i * TI, TI), :]
            dx = xj[0:1, :] - xi[:, 0:1]
            dy = xj[1:2, :] - xi[:, 1:2]
            dz = xj[2:3, :] - xi[:, 2:3]
            dist2 = dx * dx + dy * dy + dz * dz
            d2c = jnp.maximum(dist2, 1e-12)
            r0 = pl.reciprocal(d2c, approx=True)
            rcp_half = r0 * (1.0 - 0.5 * (d2c * r0))
            w = jnp.clip(_C0 - dist2 * _C1, 0.0, 1.0)
            w = jnp.where(dist2 > 0.0, w, 0.0)
            u = w * rcp_half
            mx = _copysign_pos(u, dx)
            my = _copysign_pos(u, dy)
            mz = _copysign_pos(u, dz)

            ru = jnp.dot(u, bu, preferred_element_type=jnp.float32)
            rx = jnp.dot(mx, bm[0], preferred_element_type=jnp.float32)
            ry = jnp.dot(my, bm[1], preferred_element_type=jnp.float32)
            rz = jnp.dot(mz, bm[2], preferred_element_type=jnp.float32)
            norm = jnp.dot(w, ones_n, preferred_element_type=jnp.float32)

            acc = ru[:, 0:C_BYP]
            for ax, rm in enumerate((rx, ry, rz)):
                c_i = xi[:, ax:ax + 1]
                n2 = -2.0 * c_i
                s_i = c_i * c_i
                acc = (acc
                       + n2 * ru[:, (1 + ax) * C_BYP:(2 + ax) * C_BYP]
                       + s_i * ru[:, (4 + ax) * C_BYP:(5 + ax) * C_BYP]
                       + rm[:, 0:C_BYP]
                       + n2 * rm[:, C_BYP:2 * C_BYP]
                       + s_i * rm[:, 2 * C_BYP:3 * C_BYP])
            ag_scr[pl.ds(b * N + i * TI, TI), :] = acc / jnp.maximum(norm, 1e-8)
            return _

        lax.fori_loop(0, N // TI, body, 0, unroll=True)

    ag = ag_scr[...]
    a1, sh1 = _bn_fold(ag, g1_ref[...], b1_ref[...])
    agn = jnp.maximum(ag * a1 + sh1, 0.0)
    pre = (jnp.dot(agn, wag_ref[...], preferred_element_type=jnp.float32)
           + bag_ref[...] + self_feat)
    a2, sh2 = _bn_fold(pre, g2_ref[...], b2_ref[...])
    out_ref[...] = jnp.maximum(pre * a2 + sh2, 0.0)


def kernel(feat, xyz, W_feat, b_feat, W_byp, g_byp, be_byp, W_ag, b_ag, g1, b1, g2, b2):
    xyz_c = xyz - 0.5
    out = pl.pallas_call(
        _k_all,
        out_shape=jax.ShapeDtypeStruct((B * N, C_OUT), jnp.float32),
        scratch_shapes=[pltpu.VMEM((B * N, C_BYP), jnp.float32)],
    )(feat.reshape(B * N, C_IN), xyz_c, jnp.transpose(xyz_c, (0, 2, 1)),
      W_feat.T, b_feat.reshape(1, C_OUT), W_byp.T,
      g_byp.reshape(1, C6), be_byp.reshape(1, C6),
      W_ag.T, b_ag.reshape(1, C_OUT),
      g1.reshape(1, C_BYP), b1.reshape(1, C_BYP),
      g2.reshape(1, C_OUT), b2.reshape(1, C_OUT))
    return out.reshape(B, N, C_OUT)


# speedup vs baseline: 1.3646x; 1.3646x over previous
import jax
import jax.numpy as jnp
from jax import lax
from jax.experimental import pallas as pl
from jax.experimental.pallas import tpu as pltpu

RADIUS, DECAY_RADIUS = 0.15, 0.3
EPS_BN = 1e-5
B, N = 4, 1024
C_IN, C_OUT, C_BYP = 64, 64, 32
C6 = 6 * C_BYP
TI = 256
BN_CNT = B * N
_R2 = RADIUS * RADIUS
_DR2 = DECAY_RADIUS * DECAY_RADIUS
_C1 = 1.0 / (_DR2 - _R2)
_C0 = _DR2 * _C1


def _bn_fold(x, gamma, beta):
    s1 = jnp.sum(x, axis=0, keepdims=True)
    s2 = jnp.sum(x * x, axis=0, keepdims=True)
    mean = s1 * (1.0 / BN_CNT)
    var = s2 * (1.0 / BN_CNT) - mean * mean
    a = gamma * lax.rsqrt(var + EPS_BN)
    return a, beta - mean * a


def _k_all(feat_ref, xyz_ref, xyzt_ref, wf_ref, bf_ref, wb_ref, gb_ref, beb_ref,
           wag_ref, bag_ref, g1_ref, b1_ref, g2_ref, b2_ref, out_ref, ag_scr):
    feat = feat_ref[...]
    self_feat = jnp.dot(feat, wf_ref[...], preferred_element_type=jnp.float32) + bf_ref[...]
    mut = jnp.dot(feat, wb_ref[...], preferred_element_type=jnp.float32)

    a_b, sh_b = _bn_fold(mut, gb_ref[...], beb_ref[...])

    ones_n = jnp.ones((N, C_BYP), dtype=jnp.float32)
    for b in range(B):
        g = jnp.maximum(mut[b * N:(b + 1) * N, :] * a_b + sh_b, 0.0)
        gdiff = [g[:, (2 * ax) * C_BYP:(2 * ax + 1) * C_BYP]
                 - g[:, (2 * ax + 1) * C_BYP:(2 * ax + 2) * C_BYP] for ax in range(3)]
        xj = xyzt_ref[b]

        def body(i, _):
            xi = xyz_ref[b, pl.ds(i * TI, TI), :]
            dx = xj[0:1, :] - xi[:, 0:1]
            dy = xj[1:2, :] - xi[:, 1:2]
            dz = xj[2:3, :] - xi[:, 2:3]
            sqx = dx * dx
            sqy = dy * dy
            sqz = dz * dz
            dist2 = sqx + sqy + sqz
            d2c = jnp.maximum(dist2, 1e-12)
            rcp = pl.reciprocal(d2c, approx=True)
            w = jnp.clip(_C0 - dist2 * _C1, 0.0, 1.0)
            u = w * rcp

            acc = jnp.zeros((TI, C_BYP), dtype=jnp.float32)
            for ax, (d, sq) in enumerate(((dx, sqx), (dy, sqy), (dz, sqz))):
                q = u * sq
                ap = jnp.where(d > 0.0, q, 0.0)
                acc += jnp.dot(ap, gdiff[ax], preferred_element_type=jnp.float32)
                acc += jnp.dot(q, g[:, (2 * ax + 1) * C_BYP:(2 * ax + 2) * C_BYP],
                               preferred_element_type=jnp.float32)
            norm = jnp.dot(w, ones_n, preferred_element_type=jnp.float32) - 1.0
            ag_scr[pl.ds(b * N + i * TI, TI), :] = acc / jnp.maximum(norm, 1e-8)
            return _

        lax.fori_loop(0, N // TI, body, 0, unroll=True)

    ag = ag_scr[...]
    a1, sh1 = _bn_fold(ag, g1_ref[...], b1_ref[...])
    agn = jnp.maximum(ag * a1 + sh1, 0.0)
    pre = (jnp.dot(agn, wag_ref[...], preferred_element_type=jnp.float32)
           + bag_ref[...] + self_feat)
    a2, sh2 = _bn_fold(pre, g2_ref[...], b2_ref[...])
    out_ref[...] = jnp.maximum(pre * a2 + sh2, 0.0)


def kernel(feat, xyz, W_feat, b_feat, W_byp, g_byp, be_byp, W_ag, b_ag, g1, b1, g2, b2):
    out = pl.pallas_call(
        _k_all,
        out_shape=jax.ShapeDtypeStruct((B * N, C_OUT), jnp.float32),
        scratch_shapes=[pltpu.VMEM((B * N, C_BYP), jnp.float32)],
    )(feat.reshape(B * N, C_IN), xyz, jnp.transpose(xyz, (0, 2, 1)),
      W_feat.T, b_feat.reshape(1, C_OUT), W_byp.T,
      g_byp.reshape(1, C6), be_byp.reshape(1, C6),
      W_ag.T, b_ag.reshape(1, C_OUT),
      g1.reshape(1, C_BYP), b1.reshape(1, C_BYP),
      g2.reshape(1, C_OUT), b2.reshape(1, C_OUT))
    return out.reshape(B, N, C_OUT)


# speedup vs baseline: 1.3824x; 1.0130x over previous
import jax
import jax.numpy as jnp
from jax import lax
from jax.experimental import pallas as pl
from jax.experimental.pallas import tpu as pltpu

RADIUS, DECAY_RADIUS = 0.15, 0.3
EPS_BN = 1e-5
B, N = 4, 1024
C_IN, C_OUT, C_BYP = 64, 64, 32
C6 = 6 * C_BYP
TI = 256
BN_CNT = B * N
_R2 = RADIUS * RADIUS
_DR2 = DECAY_RADIUS * DECAY_RADIUS
_C1 = 1.0 / (_DR2 - _R2)
_C0 = _DR2 * _C1


def _bn_fold(x, gamma, beta):
    s1 = jnp.sum(x, axis=0, keepdims=True)
    s2 = jnp.sum(x * x, axis=0, keepdims=True)
    mean = s1 * (1.0 / BN_CNT)
    var = s2 * (1.0 / BN_CNT) - mean * mean
    a = gamma * lax.rsqrt(var + EPS_BN)
    return a, beta - mean * a


def _k_all(feat_ref, xyz_ref, xyzt_ref, wf_ref, bf_ref, wb_ref, gb_ref, beb_ref,
           wag_ref, bag_ref, g1_ref, b1_ref, g2_ref, b2_ref, out_ref, ag_scr):
    feat = feat_ref[...]
    self_feat = jnp.dot(feat, wf_ref[...], preferred_element_type=jnp.float32) + bf_ref[...]
    mut = jnp.dot(feat, wb_ref[...], preferred_element_type=jnp.float32)

    a_b, sh_b = _bn_fold(mut, gb_ref[...], beb_ref[...])

    ones_t = jnp.ones((TI, C_BYP), dtype=jnp.float32)
    nt = N // TI
    for b in range(B):
        g = jnp.maximum(mut[b * N:(b + 1) * N, :] * a_b + sh_b, 0.0)
        gdiff = [g[:, (2 * ax) * C_BYP:(2 * ax + 1) * C_BYP]
                 - g[:, (2 * ax + 1) * C_BYP:(2 * ax + 2) * C_BYP] for ax in range(3)]
        gm = [g[:, (2 * ax + 1) * C_BYP:(2 * ax + 2) * C_BYP] for ax in range(3)]
        xj = xyzt_ref[b]

        accs = [jnp.zeros((TI, C_BYP), dtype=jnp.float32) for _ in range(nt)]
        norms = [jnp.zeros((TI, C_BYP), dtype=jnp.float32) for _ in range(nt)]
        for ii in range(nt):
            xi = xyz_ref[b, ii * TI:(ii + 1) * TI, :]
            for jj in range(ii, nt):
                xr = xj[:, jj * TI:(jj + 1) * TI]
                dx = xr[0:1, :] - xi[:, 0:1]
                dy = xr[1:2, :] - xi[:, 1:2]
                dz = xr[2:3, :] - xi[:, 2:3]
                sqx = dx * dx
                sqy = dy * dy
                sqz = dz * dz
                dist2 = sqx + sqy + sqz
                d2c = jnp.maximum(dist2, 1e-12)
                rcp = pl.reciprocal(d2c, approx=True)
                w = jnp.clip(_C0 - dist2 * _C1, 0.0, 1.0)
                u = w * rcp
                jsl = slice(jj * TI, (jj + 1) * TI)
                isl = slice(ii * TI, (ii + 1) * TI)
                for ax, (d, sq) in enumerate(((dx, sqx), (dy, sqy), (dz, sqz))):
                    q = u * sq
                    ap = jnp.where(d > 0.0, q, 0.0)
                    accs[ii] += jnp.dot(ap, gdiff[ax][jsl],
                                        preferred_element_type=jnp.float32)
                    accs[ii] += jnp.dot(q, gm[ax][jsl],
                                        preferred_element_type=jnp.float32)
                    if jj > ii:
                        qt = q.T
                        apt = qt - ap.T
                        accs[jj] += jnp.dot(apt, gdiff[ax][isl],
                                            preferred_element_type=jnp.float32)
                        accs[jj] += jnp.dot(qt, gm[ax][isl],
                                            preferred_element_type=jnp.float32)
                norms[ii] += jnp.dot(w, ones_t, preferred_element_type=jnp.float32)
                if jj > ii:
                    norms[jj] += jnp.dot(w.T, ones_t,
                                         preferred_element_type=jnp.float32)
        for ii in range(nt):
            ag_scr[b * N + ii * TI:b * N + (ii + 1) * TI, :] = (
                accs[ii] / jnp.maximum(norms[ii] - 1.0, 1e-8))

    ag = ag_scr[...]
    a1, sh1 = _bn_fold(ag, g1_ref[...], b1_ref[...])
    agn = jnp.maximum(ag * a1 + sh1, 0.0)
    pre = (jnp.dot(agn, wag_ref[...], preferred_element_type=jnp.float32)
           + bag_ref[...] + self_feat)
    a2, sh2 = _bn_fold(pre, g2_ref[...], b2_ref[...])
    out_ref[...] = jnp.maximum(pre * a2 + sh2, 0.0)


def kernel(feat, xyz, W_feat, b_feat, W_byp, g_byp, be_byp, W_ag, b_ag, g1, b1, g2, b2):
    out = pl.pallas_call(
        _k_all,
        out_shape=jax.ShapeDtypeStruct((B * N, C_OUT), jnp.float32),
        scratch_shapes=[pltpu.VMEM((B * N, C_BYP), jnp.float32)],
    )(feat.reshape(B * N, C_IN), xyz, jnp.transpose(xyz, (0, 2, 1)),
      W_feat.T, b_feat.reshape(1, C_OUT), W_byp.T,
      g_byp.reshape(1, C6), be_byp.reshape(1, C6),
      W_ag.T, b_ag.reshape(1, C_OUT),
      g1.reshape(1, C_BYP), b1.reshape(1, C_BYP),
      g2.reshape(1, C_OUT), b2.reshape(1, C_OUT))
    return out.reshape(B, N, C_OUT)
